# baseline (speedup 1.0000x reference)
import functools

import jax
import jax.numpy as jnp
from jax import lax
from jax.experimental import pallas as pl
from jax.experimental.pallas import tpu as pltpu
from jax.experimental.pallas import tpu_sc as plsc

N = 10000
NPAD = 10240
E = 320000
NSUB = 32
EPAD = 327680
EB = 128
NB = 80
HB = NB // 2
STRIPE = NPAD // 16
RB = 400
GRID_R = N // RB
EPS = 1e-5
SCW = 128


@functools.cache
def _make_spmm(nc):
  fc = SCW
  mesh = plsc.VectorSubcoreMesh(core_axis_name="c", subcore_axis_name="s")

  def body(h_hbm, srcr_hbm, dstr_hbm, out_hbm, acc, src_v, dst_v, rows0,
           rows1, sem0, sem1):
    c = lax.axis_index("c")
    s = lax.axis_index("s")

    def start_gather(ch, b, rows, sem):
      pltpu.make_async_copy(h_hbm.at[ch].at[src_v.at[b]], rows, sem).start()

    def finish_gather(ch, b, rows, sem):
      pltpu.make_async_copy(h_hbm.at[ch].at[src_v.at[b]], rows, sem).wait()
      pltpu.sync_copy(rows, acc.at[dst_v.at[b]], add=True)

    for ch in range(nc):
      def zrow(r, carry):
        for j in range(fc // 16):
          rows0[r, pl.ds(j * 16, 16)] = jnp.zeros((16,), jnp.float32)
        return carry
      lax.fori_loop(0, EB, zrow, 0)
      for t in range(STRIPE // EB):
        pltpu.sync_copy(rows0, acc.at[pl.ds(s * STRIPE + t * EB, EB)])
      plsc.subcore_barrier()

      for half in range(2):
        pltpu.sync_copy(srcr_hbm.at[c, s, pl.ds(half * HB, HB)], src_v)
        pltpu.sync_copy(dstr_hbm.at[c, s, pl.ds(half * HB, HB)], dst_v)

        def edge_batch(b, carry):
          pltpu.sync_copy(h_hbm.at[ch].at[src_v.at[b]], rows0)
          pltpu.sync_copy(rows0, acc.at[dst_v.at[b]], add=True)
          return carry
        lax.fori_loop(0, HB, edge_batch, 0)

      plsc.subcore_barrier()
      pltpu.sync_copy(acc.at[pl.ds(s * STRIPE, STRIPE)],
                      out_hbm.at[c, ch, pl.ds(s * STRIPE, STRIPE)])
      if ch + 1 < nc:
        plsc.subcore_barrier()

  return pl.kernel(
      body,
      out_type=jax.ShapeDtypeStruct((2, nc, NPAD, fc), jnp.float32),
      mesh=mesh,
      scratch_types=[
          pltpu.VMEM_SHARED((NPAD, fc), jnp.float32),
          pltpu.VMEM((HB, EB), jnp.int32),
          pltpu.VMEM((HB, EB), jnp.int32),
          pltpu.VMEM((EB, fc), jnp.float32),
          pltpu.VMEM((EB, fc), jnp.float32),
          pltpu.SemaphoreType.DMA,
          pltpu.SemaphoreType.DMA,
      ],
  )


def _dinv_body(degp_ref, dinv_ref):
  deg = degp_ref[0, 0, :, 0:1] + degp_ref[1, 0, :, 0:1] + 1.0
  dinv_ref[...] = lax.rsqrt(deg)


def _dinv(degparts):
  return pl.pallas_call(
      _dinv_body,
      grid=(GRID_R,),
      in_specs=[pl.BlockSpec((2, 1, RB, SCW), lambda r: (0, 0, r, 0))],
      out_specs=pl.BlockSpec((RB, 1), lambda r: (r, 0)),
      out_shape=jax.ShapeDtypeStruct((N, 1), jnp.float32),
  )(degparts)


def _bn_from_stats(s, stats, g, be):
  inv_n = 1.0 / N
  m = stats[0:1, :] * inv_n
  v = stats[1:2, :] * inv_n - m * m
  return g * (s - m) * lax.rsqrt(v + EPS) + be


def _store_chunks(h_ref, hp, nc, fc):
  for c in range(nc):
    chunk = hp[:, c * fc:(c + 1) * fc]
    if fc < SCW:
      chunk = jnp.pad(chunk, ((0, 0), (0, SCW - fc)))
    h_ref[c] = chunk


@functools.cache
def _make_lin0(k, f, nc, fc):
  def body(x_ref, w_ref, dinv_ref, h_ref):
    hp = dinv_ref[...] * jnp.dot(x_ref[...], w_ref[...],
                                 preferred_element_type=jnp.float32)
    _store_chunks(h_ref, hp, nc, fc)

  return pl.pallas_call(
      body,
      grid=(GRID_R,),
      in_specs=[
          pl.BlockSpec((RB, k), lambda r: (r, 0)),
          pl.BlockSpec((k, f), lambda r: (0, 0)),
          pl.BlockSpec((RB, 1), lambda r: (r, 0)),
      ],
      out_specs=pl.BlockSpec((nc, RB, SCW), lambda r: (0, r, 0)),
      out_shape=jax.ShapeDtypeStruct((nc, N, SCW), jnp.float32),
  )


@functools.cache
def _make_lin(k, f, nc, fc, want_xn):
  def body(s_ref, st_ref, g_ref, be_ref, w_ref, dinv_ref, h_ref, *xn_ref):
    xn = _bn_from_stats(s_ref[...], st_ref[...], g_ref[...], be_ref[...])
    hp = dinv_ref[...] * jnp.dot(xn, w_ref[...],
                                 preferred_element_type=jnp.float32)
    _store_chunks(h_ref, hp, nc, fc)
    if want_xn:
      xn_ref[0][...] = xn

  out_specs = [pl.BlockSpec((nc, RB, SCW), lambda r: (0, r, 0))]
  out_shape = [jax.ShapeDtypeStruct((nc, N, SCW), jnp.float32)]
  if want_xn:
    out_specs.append(pl.BlockSpec((RB, k), lambda r: (r, 0)))
    out_shape.append(jax.ShapeDtypeStruct((N, k), jnp.float32))
  return pl.pallas_call(
      body,
      grid=(GRID_R,),
      in_specs=[
          pl.BlockSpec((RB, k), lambda r: (r, 0)),
          pl.BlockSpec((2, k), lambda r: (0, 0)),
          pl.BlockSpec((1, k), lambda r: (0, 0)),
          pl.BlockSpec((1, k), lambda r: (0, 0)),
          pl.BlockSpec((k, f), lambda r: (0, 0)),
          pl.BlockSpec((RB, 1), lambda r: (r, 0)),
      ],
      out_specs=out_specs,
      out_shape=out_shape,
  )


@functools.cache
def _make_lin_skip(k1, k2, f, nc, fc):
  def body(s_ref, st_ref, g_ref, be_ref, part_ref, wt_ref, wb_ref, dinv_ref,
           h_ref):
    xn = _bn_from_stats(s_ref[...], st_ref[...], g_ref[...], be_ref[...])
    hp = jnp.dot(xn, wt_ref[...], preferred_element_type=jnp.float32)
    hp = hp + jnp.dot(part_ref[...], wb_ref[...],
                      preferred_element_type=jnp.float32)
    hp = dinv_ref[...] * hp
    _store_chunks(h_ref, hp, nc, fc)

  return pl.pallas_call(
      body,
      grid=(GRID_R,),
      in_specs=[
          pl.BlockSpec((RB, k1), lambda r: (r, 0)),
          pl.BlockSpec((2, k1), lambda r: (0, 0)),
          pl.BlockSpec((1, k1), lambda r: (0, 0)),
          pl.BlockSpec((1, k1), lambda r: (0, 0)),
          pl.BlockSpec((RB, k2), lambda r: (r, 0)),
          pl.BlockSpec((k1, f), lambda r: (0, 0)),
          pl.BlockSpec((k2, f), lambda r: (0, 0)),
          pl.BlockSpec((RB, 1), lambda r: (r, 0)),
      ],
      out_specs=pl.BlockSpec((nc, RB, SCW), lambda r: (0, r, 0)),
      out_shape=jax.ShapeDtypeStruct((nc, N, SCW), jnp.float32),
  )


@functools.cache
def _make_combine(nc, fc):
  def body(p_ref, h_ref, dinv_ref, b_ref, s_ref, st_ref):
    r = pl.program_id(1)
    z = (dinv_ref[...] *
         (p_ref[0, 0, :, :fc] + p_ref[1, 0, :, :fc] + h_ref[0, :, :fc])
         + b_ref[...])
    sv = jax.nn.sigmoid(z)
    s_ref[...] = sv

    @pl.when(r == 0)
    def _():
      st_ref[...] = jnp.zeros_like(st_ref)

    st_ref[0:1, :] += jnp.sum(sv, axis=0, keepdims=True)
    st_ref[1:2, :] += jnp.sum(sv * sv, axis=0, keepdims=True)

  f = nc * fc
  return pl.pallas_call(
      body,
      grid=(nc, GRID_R),
      in_specs=[
          pl.BlockSpec((2, 1, RB, SCW), lambda c, r: (0, c, r, 0)),
          pl.BlockSpec((1, RB, SCW), lambda c, r: (c, r, 0)),
          pl.BlockSpec((RB, 1), lambda c, r: (r, 0)),
          pl.BlockSpec((1, fc), lambda c, r: (0, c)),
      ],
      out_specs=[
          pl.BlockSpec((RB, fc), lambda c, r: (r, c)),
          pl.BlockSpec((2, fc), lambda c, r: (0, c)),
      ],
      out_shape=[
          jax.ShapeDtypeStruct((N, f), jnp.float32),
          jax.ShapeDtypeStruct((2, f), jnp.float32),
      ],
  )


def _final_body(s_ref, st_ref, g_ref, be_ref, out_ref):
  xn = _bn_from_stats(s_ref[:, 0:2], st_ref[:, 0:2], g_ref[...], be_ref[...])
  mx = jnp.max(xn, axis=1, keepdims=True)
  e = jnp.exp(xn - mx)
  out_ref[...] = e / jnp.sum(e, axis=1, keepdims=True)


def _final(s7, st7, g7, be7):
  return pl.pallas_call(
      _final_body,
      grid=(GRID_R,),
      in_specs=[
          pl.BlockSpec((RB, 16), lambda r: (r, 0)),
          pl.BlockSpec((2, 16), lambda r: (0, 0)),
          pl.BlockSpec((1, 2), lambda r: (0, 0)),
          pl.BlockSpec((1, 2), lambda r: (0, 0)),
      ],
      out_specs=pl.BlockSpec((RB, 2), lambda r: (r, 0)),
      out_shape=jax.ShapeDtypeStruct((N, 2), jnp.float32),
  )(s7, st7, g7, be7)


_CFG = [(1, 16), (1, 64), (1, 128), (2, 128), (1, 128), (1, 64), (1, 16),
        (1, 16)]


def kernel(x, edge_index,
           W0, b0, g0, be0, W1, b1, g1, be1, W2, b2, g2, be2,
           W3, b3, g3, be3, W4, b4, g4, be4, W5, b5, g5, be5,
           W6, b6, g6, be6, W7, b7, g7, be7):
  pad = EPAD - E
  srcr = jnp.concatenate(
      [edge_index[0], jnp.zeros((pad,), jnp.int32)]).reshape(2, 16, NB, EB)
  dstr = jnp.concatenate(
      [edge_index[1], jnp.full((pad,), N, jnp.int32)]).reshape(2, 16, NB, EB)

  W7p = jnp.pad(W7, ((0, 0), (0, 14)))
  b7p = jnp.pad(b7, (0, 14))

  ones_h = jnp.ones((1, N, SCW), jnp.float32)
  degparts = _make_spmm(1)(ones_h, srcr, dstr)
  dinv = _dinv(degparts)

  Ws = [W0, W1, W2, W3, W4, W5, W6, W7p]
  bs = [b0, b1, b2, b3, b4, b5, b6, b7p]
  gs = [g0, g1, g2, g3, g4, g5, g6]
  bes = [be0, be1, be2, be3, be4, be5, be6]

  s_prev, st_prev = None, None
  xns = {}
  for i in range(8):
    nc, fc = _CFG[i]
    W = Ws[i]
    if i == 0:
      hp = _make_lin0(x.shape[1], W.shape[1], nc, fc)(x, W, dinv)
    elif i <= 4:
      want_xn = i in (1, 2, 3)
      k = W.shape[0]
      outs = _make_lin(k, W.shape[1], nc, fc, want_xn)(
          s_prev, st_prev, gs[i - 1].reshape(1, k), bes[i - 1].reshape(1, k),
          W, dinv)
      if want_xn:
        hp, xns[i - 1] = outs
      else:
        hp = outs[0]
    else:
      part = xns[7 - i]
      k1 = part.shape[1]
      wt, wb = W[:k1], W[k1:]
      hp = _make_lin_skip(k1, k1, W.shape[1], nc, fc)(
          s_prev, st_prev, gs[i - 1].reshape(1, k1),
          bes[i - 1].reshape(1, k1), part, wt, wb, dinv)
    parts = _make_spmm(nc)(hp, srcr, dstr)
    s_prev, st_prev = _make_combine(nc, fc)(
        parts, hp, dinv, bs[i].reshape(1, nc * fc))

  return _final(s_prev, st_prev, g7.reshape(1, 2), be7.reshape(1, 2))


# speedup vs baseline: 3.8411x; 3.8411x over previous
import functools

import jax
import jax.numpy as jnp
from jax import lax
from jax.experimental import pallas as pl
from jax.experimental.pallas import tpu as pltpu
from jax.experimental.pallas import tpu_sc as plsc

N = 10000
NPAD = 10240
E = 320000
NSUB = 32
EPAD = 327680
EB = 128
NB = 80
HB = NB // 2
STRIPE = NPAD // 16
RB = 400
GRID_R = N // RB
EPS = 1e-5
SCW = 128


@functools.cache
def _make_spmm(nc):
  fc = SCW
  mesh = plsc.VectorSubcoreMesh(core_axis_name="c", subcore_axis_name="s")

  def body(h_hbm, srcr_hbm, dstr_hbm, out_hbm, acc, src_v, dst_v, rows0,
           rows1, sem0, sem1):
    c = lax.axis_index("c")
    s = lax.axis_index("s")

    def start_gather(ch, b, rows, sem):
      pltpu.make_async_copy(h_hbm.at[ch].at[src_v.at[b]], rows, sem).start()

    def finish_gather(ch, b, rows, sem):
      pltpu.make_async_copy(h_hbm.at[ch].at[src_v.at[b]], rows, sem).wait()
      pltpu.sync_copy(rows, acc.at[dst_v.at[b]], add=True)

    for ch in range(nc):
      def zrow(r, carry):
        for j in range(fc // 16):
          rows0[r, pl.ds(j * 16, 16)] = jnp.zeros((16,), jnp.float32)
        return carry
      lax.fori_loop(0, EB, zrow, 0)
      for t in range(STRIPE // EB):
        pltpu.sync_copy(rows0, acc.at[pl.ds(s * STRIPE + t * EB, EB)])
      plsc.subcore_barrier()

      for half in range(2):
        pltpu.sync_copy(srcr_hbm.at[c, s, pl.ds(half * HB, HB)], src_v)
        pltpu.sync_copy(dstr_hbm.at[c, s, pl.ds(half * HB, HB)], dst_v)

        start_gather(ch, 0, rows0, sem0)

        def edge_pair(i, carry):
          b = 2 * i
          start_gather(ch, b + 1, rows1, sem1)
          finish_gather(ch, b, rows0, sem0)
          start_gather(ch, b + 2, rows0, sem0)
          finish_gather(ch, b + 1, rows1, sem1)
          return carry

        lax.fori_loop(0, (HB - 2) // 2, edge_pair, 0)
        start_gather(ch, HB - 1, rows1, sem1)
        finish_gather(ch, HB - 2, rows0, sem0)
        finish_gather(ch, HB - 1, rows1, sem1)

      plsc.subcore_barrier()
      pltpu.sync_copy(acc.at[pl.ds(s * STRIPE, STRIPE)],
                      out_hbm.at[c, ch, pl.ds(s * STRIPE, STRIPE)])
      if ch + 1 < nc:
        plsc.subcore_barrier()

  return pl.kernel(
      body,
      out_type=jax.ShapeDtypeStruct((2, nc, NPAD, fc), jnp.float32),
      mesh=mesh,
      scratch_types=[
          pltpu.VMEM_SHARED((NPAD, fc), jnp.float32),
          pltpu.VMEM((HB, EB), jnp.int32),
          pltpu.VMEM((HB, EB), jnp.int32),
          pltpu.VMEM((EB, fc), jnp.float32),
          pltpu.VMEM((EB, fc), jnp.float32),
          pltpu.SemaphoreType.DMA,
          pltpu.SemaphoreType.DMA,
      ],
  )


def _dinv_body(degp_ref, dinv_ref):
  deg = degp_ref[0, 0, :, 0:1] + degp_ref[1, 0, :, 0:1] + 1.0
  dinv_ref[...] = lax.rsqrt(deg)


def _dinv(degparts):
  return pl.pallas_call(
      _dinv_body,
      grid=(GRID_R,),
      in_specs=[pl.BlockSpec((2, 1, RB, SCW), lambda r: (0, 0, r, 0))],
      out_specs=pl.BlockSpec((RB, 1), lambda r: (r, 0)),
      out_shape=jax.ShapeDtypeStruct((N, 1), jnp.float32),
  )(degparts)


def _bn_from_stats(s, stats, g, be):
  inv_n = 1.0 / N
  m = stats[0:1, :] * inv_n
  v = stats[1:2, :] * inv_n - m * m
  return g * (s - m) * lax.rsqrt(v + EPS) + be


def _store_chunks(h_ref, hp, nc, fc):
  for c in range(nc):
    chunk = hp[:, c * fc:(c + 1) * fc]
    if fc < SCW:
      chunk = jnp.pad(chunk, ((0, 0), (0, SCW - fc)))
    h_ref[c] = chunk


@functools.cache
def _make_lin0(k, f, nc, fc):
  def body(x_ref, w_ref, dinv_ref, h_ref):
    hp = dinv_ref[...] * jnp.dot(x_ref[...], w_ref[...],
                                 preferred_element_type=jnp.float32)
    _store_chunks(h_ref, hp, nc, fc)

  return pl.pallas_call(
      body,
      grid=(GRID_R,),
      in_specs=[
          pl.BlockSpec((RB, k), lambda r: (r, 0)),
          pl.BlockSpec((k, f), lambda r: (0, 0)),
          pl.BlockSpec((RB, 1), lambda r: (r, 0)),
      ],
      out_specs=pl.BlockSpec((nc, RB, SCW), lambda r: (0, r, 0)),
      out_shape=jax.ShapeDtypeStruct((nc, N, SCW), jnp.float32),
  )


@functools.cache
def _make_lin(k, f, nc, fc, want_xn):
  def body(s_ref, st_ref, g_ref, be_ref, w_ref, dinv_ref, h_ref, *xn_ref):
    xn = _bn_from_stats(s_ref[...], st_ref[...], g_ref[...], be_ref[...])
    hp = dinv_ref[...] * jnp.dot(xn, w_ref[...],
                                 preferred_element_type=jnp.float32)
    _store_chunks(h_ref, hp, nc, fc)
    if want_xn:
      xn_ref[0][...] = xn

  out_specs = [pl.BlockSpec((nc, RB, SCW), lambda r: (0, r, 0))]
  out_shape = [jax.ShapeDtypeStruct((nc, N, SCW), jnp.float32)]
  if want_xn:
    out_specs.append(pl.BlockSpec((RB, k), lambda r: (r, 0)))
    out_shape.append(jax.ShapeDtypeStruct((N, k), jnp.float32))
  return pl.pallas_call(
      body,
      grid=(GRID_R,),
      in_specs=[
          pl.BlockSpec((RB, k), lambda r: (r, 0)),
          pl.BlockSpec((2, k), lambda r: (0, 0)),
          pl.BlockSpec((1, k), lambda r: (0, 0)),
          pl.BlockSpec((1, k), lambda r: (0, 0)),
          pl.BlockSpec((k, f), lambda r: (0, 0)),
          pl.BlockSpec((RB, 1), lambda r: (r, 0)),
      ],
      out_specs=out_specs,
      out_shape=out_shape,
  )


@functools.cache
def _make_lin_skip(k1, k2, f, nc, fc):
  def body(s_ref, st_ref, g_ref, be_ref, part_ref, wt_ref, wb_ref, dinv_ref,
           h_ref):
    xn = _bn_from_stats(s_ref[...], st_ref[...], g_ref[...], be_ref[...])
    hp = jnp.dot(xn, wt_ref[...], preferred_element_type=jnp.float32)
    hp = hp + jnp.dot(part_ref[...], wb_ref[...],
                      preferred_element_type=jnp.float32)
    hp = dinv_ref[...] * hp
    _store_chunks(h_ref, hp, nc, fc)

  return pl.pallas_call(
      body,
      grid=(GRID_R,),
      in_specs=[
          pl.BlockSpec((RB, k1), lambda r: (r, 0)),
          pl.BlockSpec((2, k1), lambda r: (0, 0)),
          pl.BlockSpec((1, k1), lambda r: (0, 0)),
          pl.BlockSpec((1, k1), lambda r: (0, 0)),
          pl.BlockSpec((RB, k2), lambda r: (r, 0)),
          pl.BlockSpec((k1, f), lambda r: (0, 0)),
          pl.BlockSpec((k2, f), lambda r: (0, 0)),
          pl.BlockSpec((RB, 1), lambda r: (r, 0)),
      ],
      out_specs=pl.BlockSpec((nc, RB, SCW), lambda r: (0, r, 0)),
      out_shape=jax.ShapeDtypeStruct((nc, N, SCW), jnp.float32),
  )


@functools.cache
def _make_combine(nc, fc):
  def body(p_ref, h_ref, dinv_ref, b_ref, s_ref, st_ref):
    r = pl.program_id(1)
    z = (dinv_ref[...] *
         (p_ref[0, 0, :, :fc] + p_ref[1, 0, :, :fc] + h_ref[0, :, :fc])
         + b_ref[...])
    sv = jax.nn.sigmoid(z)
    s_ref[...] = sv

    @pl.when(r == 0)
    def _():
      st_ref[...] = jnp.zeros_like(st_ref)

    st_ref[0:1, :] += jnp.sum(sv, axis=0, keepdims=True)
    st_ref[1:2, :] += jnp.sum(sv * sv, axis=0, keepdims=True)

  f = nc * fc
  return pl.pallas_call(
      body,
      grid=(nc, GRID_R),
      in_specs=[
          pl.BlockSpec((2, 1, RB, SCW), lambda c, r: (0, c, r, 0)),
          pl.BlockSpec((1, RB, SCW), lambda c, r: (c, r, 0)),
          pl.BlockSpec((RB, 1), lambda c, r: (r, 0)),
          pl.BlockSpec((1, fc), lambda c, r: (0, c)),
      ],
      out_specs=[
          pl.BlockSpec((RB, fc), lambda c, r: (r, c)),
          pl.BlockSpec((2, fc), lambda c, r: (0, c)),
      ],
      out_shape=[
          jax.ShapeDtypeStruct((N, f), jnp.float32),
          jax.ShapeDtypeStruct((2, f), jnp.float32),
      ],
  )


def _final_body(s_ref, st_ref, g_ref, be_ref, out_ref):
  xn = _bn_from_stats(s_ref[:, 0:2], st_ref[:, 0:2], g_ref[...], be_ref[...])
  mx = jnp.max(xn, axis=1, keepdims=True)
  e = jnp.exp(xn - mx)
  out_ref[...] = e / jnp.sum(e, axis=1, keepdims=True)


def _final(s7, st7, g7, be7):
  return pl.pallas_call(
      _final_body,
      grid=(GRID_R,),
      in_specs=[
          pl.BlockSpec((RB, 16), lambda r: (r, 0)),
          pl.BlockSpec((2, 16), lambda r: (0, 0)),
          pl.BlockSpec((1, 2), lambda r: (0, 0)),
          pl.BlockSpec((1, 2), lambda r: (0, 0)),
      ],
      out_specs=pl.BlockSpec((RB, 2), lambda r: (r, 0)),
      out_shape=jax.ShapeDtypeStruct((N, 2), jnp.float32),
  )(s7, st7, g7, be7)


_CFG = [(1, 16), (1, 64), (1, 128), (2, 128), (1, 128), (1, 64), (1, 16),
        (1, 16)]


def kernel(x, edge_index,
           W0, b0, g0, be0, W1, b1, g1, be1, W2, b2, g2, be2,
           W3, b3, g3, be3, W4, b4, g4, be4, W5, b5, g5, be5,
           W6, b6, g6, be6, W7, b7, g7, be7):
  pad = EPAD - E
  pad_iota = jnp.arange(pad, dtype=jnp.int32)
  srcr = jnp.concatenate(
      [edge_index[0], pad_iota % N]).reshape(2, 16, NB, EB)
  dstr = jnp.concatenate(
      [edge_index[1], N + pad_iota % (NPAD - N)]).reshape(2, 16, NB, EB)

  W7p = jnp.pad(W7, ((0, 0), (0, 14)))
  b7p = jnp.pad(b7, (0, 14))

  ones_h = jnp.ones((1, N, SCW), jnp.float32)
  degparts = _make_spmm(1)(ones_h, srcr, dstr)
  dinv = _dinv(degparts)

  Ws = [W0, W1, W2, W3, W4, W5, W6, W7p]
  bs = [b0, b1, b2, b3, b4, b5, b6, b7p]
  gs = [g0, g1, g2, g3, g4, g5, g6]
  bes = [be0, be1, be2, be3, be4, be5, be6]

  s_prev, st_prev = None, None
  xns = {}
  for i in range(8):
    nc, fc = _CFG[i]
    W = Ws[i]
    if i == 0:
      hp = _make_lin0(x.shape[1], W.shape[1], nc, fc)(x, W, dinv)
    elif i <= 4:
      want_xn = i in (1, 2, 3)
      k = W.shape[0]
      outs = _make_lin(k, W.shape[1], nc, fc, want_xn)(
          s_prev, st_prev, gs[i - 1].reshape(1, k), bes[i - 1].reshape(1, k),
          W, dinv)
      if want_xn:
        hp, xns[i - 1] = outs
      else:
        hp = outs[0]
    else:
      part = xns[7 - i]
      k1 = part.shape[1]
      wt, wb = W[:k1], W[k1:]
      hp = _make_lin_skip(k1, k1, W.shape[1], nc, fc)(
          s_prev, st_prev, gs[i - 1].reshape(1, k1),
          bes[i - 1].reshape(1, k1), part, wt, wb, dinv)
    parts = _make_spmm(nc)(hp, srcr, dstr)
    s_prev, st_prev = _make_combine(nc, fc)(
        parts, hp, dinv, bs[i].reshape(1, nc * fc))

  return _final(s_prev, st_prev, g7.reshape(1, 2), be7.reshape(1, 2))


# speedup vs baseline: 3.9499x; 1.0283x over previous
import functools

import jax
import jax.numpy as jnp
from jax import lax
from jax.experimental import pallas as pl
from jax.experimental.pallas import tpu as pltpu
from jax.experimental.pallas import tpu_sc as plsc

N = 10000
NPAD = 10240
E = 320000
NSUB = 32
EPAD = 327680
EB = 128
NB = 80
HB = NB // 2
STRIPE = NPAD // 16
RB = 400
GRID_R = N // RB
EPS = 1e-5
SCW = 128


@functools.cache
def _make_spmm(nc):
  fc = SCW
  mesh = plsc.VectorSubcoreMesh(core_axis_name="c", subcore_axis_name="s")

  def body(h_hbm, srcr_hbm, dstr_hbm, out_hbm, acc, src_v, dst_v, rows0,
           rows1, sem0, sem1):
    c = lax.axis_index("c")
    s = lax.axis_index("s")

    def start_gather(ch, b, rows, sem):
      pltpu.make_async_copy(h_hbm.at[ch].at[src_v.at[b]], rows, sem).start()

    def finish_gather(ch, b, rows, sem):
      pltpu.make_async_copy(h_hbm.at[ch].at[src_v.at[b]], rows, sem).wait()
      pltpu.sync_copy(rows, acc.at[dst_v.at[b]], add=True)

    for ch in range(nc):
      def zrow(r, carry):
        for j in range(fc // 16):
          rows0[r, pl.ds(j * 16, 16)] = jnp.zeros((16,), jnp.float32)
        return carry
      lax.fori_loop(0, EB, zrow, 0)
      for t in range(STRIPE // EB):
        pltpu.sync_copy(rows0, acc.at[pl.ds(s * STRIPE + t * EB, EB)])
      plsc.subcore_barrier()

      for half in range(2):
        pltpu.sync_copy(srcr_hbm.at[c, s, pl.ds(half * HB, HB)], src_v)
        pltpu.sync_copy(dstr_hbm.at[c, s, pl.ds(half * HB, HB)], dst_v)

        start_gather(ch, 0, rows0, sem0)

        def edge_pair(i, carry):
          b = 2 * i
          start_gather(ch, b + 1, rows1, sem1)
          finish_gather(ch, b, rows0, sem0)
          start_gather(ch, b + 2, rows0, sem0)
          finish_gather(ch, b + 1, rows1, sem1)
          return carry

        lax.fori_loop(0, (HB - 2) // 2, edge_pair, 0)
        start_gather(ch, HB - 1, rows1, sem1)
        finish_gather(ch, HB - 2, rows0, sem0)
        finish_gather(ch, HB - 1, rows1, sem1)

      plsc.subcore_barrier()
      pltpu.sync_copy(acc.at[pl.ds(s * STRIPE, STRIPE)],
                      out_hbm.at[c, ch, pl.ds(s * STRIPE, STRIPE)])
      if ch + 1 < nc:
        plsc.subcore_barrier()

  return pl.kernel(
      body,
      out_type=jax.ShapeDtypeStruct((2, nc, NPAD, fc), jnp.float32),
      mesh=mesh,
      scratch_types=[
          pltpu.VMEM_SHARED((NPAD, fc), jnp.float32),
          pltpu.VMEM((HB, EB), jnp.int32),
          pltpu.VMEM((HB, EB), jnp.int32),
          pltpu.VMEM((EB, fc), jnp.float32),
          pltpu.VMEM((EB, fc), jnp.float32),
          pltpu.SemaphoreType.DMA,
          pltpu.SemaphoreType.DMA,
      ],
  )


@functools.cache
def _make_deg():
  mesh = plsc.VectorSubcoreMesh(core_axis_name="c", subcore_axis_name="s")

  def body(dstr_hbm, out_hbm, acc, dst_v, rows0):
    c = lax.axis_index("c")
    s = lax.axis_index("s")

    def fill(val):
      def frow(r, carry):
        for j in range(SCW // 16):
          rows0[r, pl.ds(j * 16, 16)] = jnp.full((16,), val, jnp.float32)
        return carry
      lax.fori_loop(0, EB, frow, 0)

    fill(0.0)
    for t in range(STRIPE // EB):
      pltpu.sync_copy(rows0, acc.at[pl.ds(s * STRIPE + t * EB, EB)])
    plsc.subcore_barrier()
    fill(1.0)

    for half in range(2):
      pltpu.sync_copy(dstr_hbm.at[c, s, pl.ds(half * HB, HB)], dst_v)

      def batch(b, carry):
        pltpu.sync_copy(rows0, acc.at[dst_v.at[b]], add=True)
        return carry
      lax.fori_loop(0, HB, batch, 0)

    plsc.subcore_barrier()
    pltpu.sync_copy(acc.at[pl.ds(s * STRIPE, STRIPE)],
                    out_hbm.at[c, pl.ds(s * STRIPE, STRIPE)])

  return pl.kernel(
      body,
      out_type=jax.ShapeDtypeStruct((2, NPAD, SCW), jnp.float32),
      mesh=mesh,
      scratch_types=[
          pltpu.VMEM_SHARED((NPAD, SCW), jnp.float32),
          pltpu.VMEM((HB, EB), jnp.int32),
          pltpu.VMEM((EB, SCW), jnp.float32),
      ],
  )


def _dinv_body(degp_ref, dinv_ref):
  deg = degp_ref[0, :, 0:1] + degp_ref[1, :, 0:1] + 1.0
  dinv_ref[...] = lax.rsqrt(deg)


def _dinv(degparts):
  return pl.pallas_call(
      _dinv_body,
      grid=(GRID_R,),
      in_specs=[pl.BlockSpec((2, RB, SCW), lambda r: (0, r, 0))],
      out_specs=pl.BlockSpec((RB, 1), lambda r: (r, 0)),
      out_shape=jax.ShapeDtypeStruct((N, 1), jnp.float32),
  )(degparts)


def _bn_from_stats(s, stats, g, be):
  inv_n = 1.0 / N
  m = stats[0:1, :] * inv_n
  v = stats[1:2, :] * inv_n - m * m
  return g * (s - m) * lax.rsqrt(v + EPS) + be


def _store_chunks(h_ref, hp, nc, fc):
  for c in range(nc):
    chunk = hp[:, c * fc:(c + 1) * fc]
    if fc < SCW:
      chunk = jnp.pad(chunk, ((0, 0), (0, SCW - fc)))
    h_ref[c] = chunk


@functools.cache
def _make_lin0(k, f, nc, fc):
  def body(x_ref, w_ref, dinv_ref, h_ref):
    hp = dinv_ref[...] * jnp.dot(x_ref[...], w_ref[...],
                                 preferred_element_type=jnp.float32)
    _store_chunks(h_ref, hp, nc, fc)

  return pl.pallas_call(
      body,
      grid=(GRID_R,),
      in_specs=[
          pl.BlockSpec((RB, k), lambda r: (r, 0)),
          pl.BlockSpec((k, f), lambda r: (0, 0)),
          pl.BlockSpec((RB, 1), lambda r: (r, 0)),
      ],
      out_specs=pl.BlockSpec((nc, RB, SCW), lambda r: (0, r, 0)),
      out_shape=jax.ShapeDtypeStruct((nc, N, SCW), jnp.float32),
  )


@functools.cache
def _make_lin(k, f, nc, fc, want_xn):
  def body(s_ref, st_ref, g_ref, be_ref, w_ref, dinv_ref, h_ref, *xn_ref):
    xn = _bn_from_stats(s_ref[...], st_ref[...], g_ref[...], be_ref[...])
    hp = dinv_ref[...] * jnp.dot(xn, w_ref[...],
                                 preferred_element_type=jnp.float32)
    _store_chunks(h_ref, hp, nc, fc)
    if want_xn:
      xn_ref[0][...] = xn

  out_specs = [pl.BlockSpec((nc, RB, SCW), lambda r: (0, r, 0))]
  out_shape = [jax.ShapeDtypeStruct((nc, N, SCW), jnp.float32)]
  if want_xn:
    out_specs.append(pl.BlockSpec((RB, k), lambda r: (r, 0)))
    out_shape.append(jax.ShapeDtypeStruct((N, k), jnp.float32))
  return pl.pallas_call(
      body,
      grid=(GRID_R,),
      in_specs=[
          pl.BlockSpec((RB, k), lambda r: (r, 0)),
          pl.BlockSpec((2, k), lambda r: (0, 0)),
          pl.BlockSpec((1, k), lambda r: (0, 0)),
          pl.BlockSpec((1, k), lambda r: (0, 0)),
          pl.BlockSpec((k, f), lambda r: (0, 0)),
          pl.BlockSpec((RB, 1), lambda r: (r, 0)),
      ],
      out_specs=out_specs,
      out_shape=out_shape,
  )


@functools.cache
def _make_lin_skip(k1, k2, f, nc, fc):
  def body(s_ref, st_ref, g_ref, be_ref, part_ref, wt_ref, wb_ref, dinv_ref,
           h_ref):
    xn = _bn_from_stats(s_ref[...], st_ref[...], g_ref[...], be_ref[...])
    hp = jnp.dot(xn, wt_ref[...], preferred_element_type=jnp.float32)
    hp = hp + jnp.dot(part_ref[...], wb_ref[...],
                      preferred_element_type=jnp.float32)
    hp = dinv_ref[...] * hp
    _store_chunks(h_ref, hp, nc, fc)

  return pl.pallas_call(
      body,
      grid=(GRID_R,),
      in_specs=[
          pl.BlockSpec((RB, k1), lambda r: (r, 0)),
          pl.BlockSpec((2, k1), lambda r: (0, 0)),
          pl.BlockSpec((1, k1), lambda r: (0, 0)),
          pl.BlockSpec((1, k1), lambda r: (0, 0)),
          pl.BlockSpec((RB, k2), lambda r: (r, 0)),
          pl.BlockSpec((k1, f), lambda r: (0, 0)),
          pl.BlockSpec((k2, f), lambda r: (0, 0)),
          pl.BlockSpec((RB, 1), lambda r: (r, 0)),
      ],
      out_specs=pl.BlockSpec((nc, RB, SCW), lambda r: (0, r, 0)),
      out_shape=jax.ShapeDtypeStruct((nc, N, SCW), jnp.float32),
  )


@functools.cache
def _make_combine(nc, fc):
  def body(p_ref, h_ref, dinv_ref, b_ref, s_ref, st_ref):
    r = pl.program_id(1)
    z = (dinv_ref[...] *
         (p_ref[0, 0, :, :fc] + p_ref[1, 0, :, :fc] + h_ref[0, :, :fc])
         + b_ref[...])
    sv = jax.nn.sigmoid(z)
    s_ref[...] = sv

    @pl.when(r == 0)
    def _():
      st_ref[...] = jnp.zeros_like(st_ref)

    st_ref[0:1, :] += jnp.sum(sv, axis=0, keepdims=True)
    st_ref[1:2, :] += jnp.sum(sv * sv, axis=0, keepdims=True)

  f = nc * fc
  return pl.pallas_call(
      body,
      grid=(nc, GRID_R),
      in_specs=[
          pl.BlockSpec((2, 1, RB, SCW), lambda c, r: (0, c, r, 0)),
          pl.BlockSpec((1, RB, SCW), lambda c, r: (c, r, 0)),
          pl.BlockSpec((RB, 1), lambda c, r: (r, 0)),
          pl.BlockSpec((1, fc), lambda c, r: (0, c)),
      ],
      out_specs=[
          pl.BlockSpec((RB, fc), lambda c, r: (r, c)),
          pl.BlockSpec((2, fc), lambda c, r: (0, c)),
      ],
      out_shape=[
          jax.ShapeDtypeStruct((N, f), jnp.float32),
          jax.ShapeDtypeStruct((2, f), jnp.float32),
      ],
  )


def _final_body(s_ref, st_ref, g_ref, be_ref, out_ref):
  xn = _bn_from_stats(s_ref[:, 0:2], st_ref[:, 0:2], g_ref[...], be_ref[...])
  mx = jnp.max(xn, axis=1, keepdims=True)
  e = jnp.exp(xn - mx)
  out_ref[...] = e / jnp.sum(e, axis=1, keepdims=True)


def _final(s7, st7, g7, be7):
  return pl.pallas_call(
      _final_body,
      grid=(GRID_R,),
      in_specs=[
          pl.BlockSpec((RB, 16), lambda r: (r, 0)),
          pl.BlockSpec((2, 16), lambda r: (0, 0)),
          pl.BlockSpec((1, 2), lambda r: (0, 0)),
          pl.BlockSpec((1, 2), lambda r: (0, 0)),
      ],
      out_specs=pl.BlockSpec((RB, 2), lambda r: (r, 0)),
      out_shape=jax.ShapeDtypeStruct((N, 2), jnp.float32),
  )(s7, st7, g7, be7)


_CFG = [(1, 16), (1, 64), (1, 128), (2, 128), (1, 128), (1, 64), (1, 16),
        (1, 16)]


def kernel(x, edge_index,
           W0, b0, g0, be0, W1, b1, g1, be1, W2, b2, g2, be2,
           W3, b3, g3, be3, W4, b4, g4, be4, W5, b5, g5, be5,
           W6, b6, g6, be6, W7, b7, g7, be7):
  pad = EPAD - E
  pad_iota = jnp.arange(pad, dtype=jnp.int32)
  srcr = jnp.concatenate(
      [edge_index[0], pad_iota % N]).reshape(2, 16, NB, EB)
  dstr = jnp.concatenate(
      [edge_index[1], N + pad_iota % (NPAD - N)]).reshape(2, 16, NB, EB)

  W7p = jnp.pad(W7, ((0, 0), (0, 14)))
  b7p = jnp.pad(b7, (0, 14))

  degparts = _make_deg()(dstr)
  dinv = _dinv(degparts)

  Ws = [W0, W1, W2, W3, W4, W5, W6, W7p]
  bs = [b0, b1, b2, b3, b4, b5, b6, b7p]
  gs = [g0, g1, g2, g3, g4, g5, g6]
  bes = [be0, be1, be2, be3, be4, be5, be6]

  s_prev, st_prev = None, None
  xns = {}
  for i in range(8):
    nc, fc = _CFG[i]
    W = Ws[i]
    if i == 0:
      hp = _make_lin0(x.shape[1], W.shape[1], nc, fc)(x, W, dinv)
    elif i <= 4:
      want_xn = i in (1, 2, 3)
      k = W.shape[0]
      outs = _make_lin(k, W.shape[1], nc, fc, want_xn)(
          s_prev, st_prev, gs[i - 1].reshape(1, k), bes[i - 1].reshape(1, k),
          W, dinv)
      if want_xn:
        hp, xns[i - 1] = outs
      else:
        hp = outs[0]
    else:
      part = xns[7 - i]
      k1 = part.shape[1]
      wt, wb = W[:k1], W[k1:]
      hp = _make_lin_skip(k1, k1, W.shape[1], nc, fc)(
          s_prev, st_prev, gs[i - 1].reshape(1, k1),
          bes[i - 1].reshape(1, k1), part, wt, wb, dinv)
    parts = _make_spmm(nc)(hp, srcr, dstr)
    s_prev, st_prev = _make_combine(nc, fc)(
        parts, hp, dinv, bs[i].reshape(1, nc * fc))

  return _final(s_prev, st_prev, g7.reshape(1, 2), be7.reshape(1, 2))


# speedup vs baseline: 4.3801x; 1.1089x over previous
import functools

import jax
import jax.numpy as jnp
from jax import lax
from jax.experimental import pallas as pl
from jax.experimental.pallas import tpu as pltpu
from jax.experimental.pallas import tpu_sc as plsc

N = 10000
NPAD = 10240
E = 320000
NSUB = 32
EPAD = 327680
EB = 128
NB = 80
HB = NB // 2
STRIPE = NPAD // 16
RB = 1000
GRID_R = N // RB
EPS = 1e-5
SCW = 128


@functools.cache
def _make_spmm(nc):
  fc = SCW
  mesh = plsc.VectorSubcoreMesh(core_axis_name="c", subcore_axis_name="s")

  def body(h_hbm, srcr_hbm, dstr_hbm, out_hbm, acc, src_v, dst_v, rows0,
           rows1, sem0, sem1):
    c = lax.axis_index("c")
    s = lax.axis_index("s")

    def start_gather(ch, b, rows, sem):
      pltpu.make_async_copy(h_hbm.at[ch].at[src_v.at[b]], rows, sem).start()

    def finish_gather(ch, b, rows, sem):
      pltpu.make_async_copy(h_hbm.at[ch].at[src_v.at[b]], rows, sem).wait()
      pltpu.sync_copy(rows, acc.at[dst_v.at[b]], add=True)

    for ch in range(nc):
      def zrow(r, carry):
        for j in range(fc // 16):
          rows0[r, pl.ds(j * 16, 16)] = jnp.zeros((16,), jnp.float32)
        return carry
      lax.fori_loop(0, EB, zrow, 0)
      for t in range(STRIPE // EB):
        pltpu.sync_copy(rows0, acc.at[pl.ds(s * STRIPE + t * EB, EB)])
      plsc.subcore_barrier()

      for half in range(2):
        pltpu.sync_copy(srcr_hbm.at[c, s, pl.ds(half * HB, HB)], src_v)
        pltpu.sync_copy(dstr_hbm.at[c, s, pl.ds(half * HB, HB)], dst_v)

        start_gather(ch, 0, rows0, sem0)

        def edge_pair(i, carry):
          b = 2 * i
          start_gather(ch, b + 1, rows1, sem1)
          finish_gather(ch, b, rows0, sem0)
          start_gather(ch, b + 2, rows0, sem0)
          finish_gather(ch, b + 1, rows1, sem1)
          return carry

        lax.fori_loop(0, (HB - 2) // 2, edge_pair, 0)
        start_gather(ch, HB - 1, rows1, sem1)
        finish_gather(ch, HB - 2, rows0, sem0)
        finish_gather(ch, HB - 1, rows1, sem1)

      plsc.subcore_barrier()
      pltpu.sync_copy(acc.at[pl.ds(s * STRIPE, STRIPE)],
                      out_hbm.at[c, ch, pl.ds(s * STRIPE, STRIPE)])
      if ch + 1 < nc:
        plsc.subcore_barrier()

  return pl.kernel(
      body,
      out_type=jax.ShapeDtypeStruct((2, nc, NPAD, fc), jnp.float32),
      mesh=mesh,
      scratch_types=[
          pltpu.VMEM_SHARED((NPAD, fc), jnp.float32),
          pltpu.VMEM((HB, EB), jnp.int32),
          pltpu.VMEM((HB, EB), jnp.int32),
          pltpu.VMEM((EB, fc), jnp.float32),
          pltpu.VMEM((EB, fc), jnp.float32),
          pltpu.SemaphoreType.DMA,
          pltpu.SemaphoreType.DMA,
      ],
  )


@functools.cache
def _make_deg():
  mesh = plsc.VectorSubcoreMesh(core_axis_name="c", subcore_axis_name="s")

  def body(dstr_hbm, out_hbm, acc, dst_v, rows0):
    c = lax.axis_index("c")
    s = lax.axis_index("s")

    def fill(val):
      def frow(r, carry):
        for j in range(SCW // 16):
          rows0[r, pl.ds(j * 16, 16)] = jnp.full((16,), val, jnp.float32)
        return carry
      lax.fori_loop(0, EB, frow, 0)

    fill(0.0)
    for t in range(STRIPE // EB):
      pltpu.sync_copy(rows0, acc.at[pl.ds(s * STRIPE + t * EB, EB)])
    plsc.subcore_barrier()
    fill(1.0)

    for half in range(2):
      pltpu.sync_copy(dstr_hbm.at[c, s, pl.ds(half * HB, HB)], dst_v)

      def batch(b, carry):
        pltpu.sync_copy(rows0, acc.at[dst_v.at[b]], add=True)
        return carry
      lax.fori_loop(0, HB, batch, 0)

    plsc.subcore_barrier()
    pltpu.sync_copy(acc.at[pl.ds(s * STRIPE, STRIPE)],
                    out_hbm.at[c, pl.ds(s * STRIPE, STRIPE)])

  return pl.kernel(
      body,
      out_type=jax.ShapeDtypeStruct((2, NPAD, SCW), jnp.float32),
      mesh=mesh,
      scratch_types=[
          pltpu.VMEM_SHARED((NPAD, SCW), jnp.float32),
          pltpu.VMEM((HB, EB), jnp.int32),
          pltpu.VMEM((EB, SCW), jnp.float32),
      ],
  )


def _dinv_body(degp_ref, dinv_ref):
  deg = degp_ref[0, :, 0:1] + degp_ref[1, :, 0:1] + 1.0
  dinv_ref[...] = lax.rsqrt(deg)


def _dinv(degparts):
  return pl.pallas_call(
      _dinv_body,
      grid=(GRID_R,),
      in_specs=[pl.BlockSpec((2, RB, SCW), lambda r: (0, r, 0))],
      out_specs=pl.BlockSpec((RB, 1), lambda r: (r, 0)),
      out_shape=jax.ShapeDtypeStruct((N, 1), jnp.float32),
  )(degparts)


def _bn_from_stats(s, stats, g, be):
  inv_n = 1.0 / N
  m = stats[0:1, :] * inv_n
  v = stats[1:2, :] * inv_n - m * m
  return g * (s - m) * lax.rsqrt(v + EPS) + be


def _store_chunks(h_ref, hp, nc, fc):
  for c in range(nc):
    chunk = hp[:, c * fc:(c + 1) * fc]
    if fc < SCW:
      chunk = jnp.pad(chunk, ((0, 0), (0, SCW - fc)))
    h_ref[c] = chunk


@functools.cache
def _make_lin0(k, f, nc, fc):
  def body(x_ref, w_ref, dinv_ref, h_ref):
    hp = dinv_ref[...] * jnp.dot(x_ref[...], w_ref[...],
                                 preferred_element_type=jnp.float32)
    _store_chunks(h_ref, hp, nc, fc)

  return pl.pallas_call(
      body,
      grid=(GRID_R,),
      in_specs=[
          pl.BlockSpec((RB, k), lambda r: (r, 0)),
          pl.BlockSpec((k, f), lambda r: (0, 0)),
          pl.BlockSpec((RB, 1), lambda r: (r, 0)),
      ],
      out_specs=pl.BlockSpec((nc, RB, SCW), lambda r: (0, r, 0)),
      out_shape=jax.ShapeDtypeStruct((nc, N, SCW), jnp.float32),
  )


@functools.cache
def _make_lin(k, f, nc, fc, want_xn):
  def body(s_ref, st_ref, g_ref, be_ref, w_ref, dinv_ref, h_ref, *xn_ref):
    xn = _bn_from_stats(s_ref[...], st_ref[...], g_ref[...], be_ref[...])
    hp = dinv_ref[...] * jnp.dot(xn, w_ref[...],
                                 preferred_element_type=jnp.float32)
    _store_chunks(h_ref, hp, nc, fc)
    if want_xn:
      xn_ref[0][...] = xn

  out_specs = [pl.BlockSpec((nc, RB, SCW), lambda r: (0, r, 0))]
  out_shape = [jax.ShapeDtypeStruct((nc, N, SCW), jnp.float32)]
  if want_xn:
    out_specs.append(pl.BlockSpec((RB, k), lambda r: (r, 0)))
    out_shape.append(jax.ShapeDtypeStruct((N, k), jnp.float32))
  return pl.pallas_call(
      body,
      grid=(GRID_R,),
      in_specs=[
          pl.BlockSpec((RB, k), lambda r: (r, 0)),
          pl.BlockSpec((2, k), lambda r: (0, 0)),
          pl.BlockSpec((1, k), lambda r: (0, 0)),
          pl.BlockSpec((1, k), lambda r: (0, 0)),
          pl.BlockSpec((k, f), lambda r: (0, 0)),
          pl.BlockSpec((RB, 1), lambda r: (r, 0)),
      ],
      out_specs=out_specs,
      out_shape=out_shape,
  )


@functools.cache
def _make_lin_skip(k1, k2, f, nc, fc):
  def body(s_ref, st_ref, g_ref, be_ref, part_ref, wt_ref, wb_ref, dinv_ref,
           h_ref):
    xn = _bn_from_stats(s_ref[...], st_ref[...], g_ref[...], be_ref[...])
    hp = jnp.dot(xn, wt_ref[...], preferred_element_type=jnp.float32)
    hp = hp + jnp.dot(part_ref[...], wb_ref[...],
                      preferred_element_type=jnp.float32)
    hp = dinv_ref[...] * hp
    _store_chunks(h_ref, hp, nc, fc)

  return pl.pallas_call(
      body,
      grid=(GRID_R,),
      in_specs=[
          pl.BlockSpec((RB, k1), lambda r: (r, 0)),
          pl.BlockSpec((2, k1), lambda r: (0, 0)),
          pl.BlockSpec((1, k1), lambda r: (0, 0)),
          pl.BlockSpec((1, k1), lambda r: (0, 0)),
          pl.BlockSpec((RB, k2), lambda r: (r, 0)),
          pl.BlockSpec((k1, f), lambda r: (0, 0)),
          pl.BlockSpec((k2, f), lambda r: (0, 0)),
          pl.BlockSpec((RB, 1), lambda r: (r, 0)),
      ],
      out_specs=pl.BlockSpec((nc, RB, SCW), lambda r: (0, r, 0)),
      out_shape=jax.ShapeDtypeStruct((nc, N, SCW), jnp.float32),
  )


@functools.cache
def _make_combine(nc, fc):
  def body(p_ref, h_ref, dinv_ref, b_ref, s_ref, st_ref):
    r = pl.program_id(1)
    z = (dinv_ref[...] *
         (p_ref[0, 0, :, :fc] + p_ref[1, 0, :, :fc] + h_ref[0, :, :fc])
         + b_ref[...])
    sv = jax.nn.sigmoid(z)
    s_ref[...] = sv

    @pl.when(r == 0)
    def _():
      st_ref[...] = jnp.zeros_like(st_ref)

    st_ref[0:1, :] += jnp.sum(sv, axis=0, keepdims=True)
    st_ref[1:2, :] += jnp.sum(sv * sv, axis=0, keepdims=True)

  f = nc * fc
  return pl.pallas_call(
      body,
      grid=(nc, GRID_R),
      in_specs=[
          pl.BlockSpec((2, 1, RB, SCW), lambda c, r: (0, c, r, 0)),
          pl.BlockSpec((1, RB, SCW), lambda c, r: (c, r, 0)),
          pl.BlockSpec((RB, 1), lambda c, r: (r, 0)),
          pl.BlockSpec((1, fc), lambda c, r: (0, c)),
      ],
      out_specs=[
          pl.BlockSpec((RB, fc), lambda c, r: (r, c)),
          pl.BlockSpec((2, fc), lambda c, r: (0, c)),
      ],
      out_shape=[
          jax.ShapeDtypeStruct((N, f), jnp.float32),
          jax.ShapeDtypeStruct((2, f), jnp.float32),
      ],
  )


def _final_body(s_ref, st_ref, g_ref, be_ref, out_ref):
  xn = _bn_from_stats(s_ref[:, 0:2], st_ref[:, 0:2], g_ref[...], be_ref[...])
  mx = jnp.max(xn, axis=1, keepdims=True)
  e = jnp.exp(xn - mx)
  out_ref[...] = e / jnp.sum(e, axis=1, keepdims=True)


def _final(s7, st7, g7, be7):
  return pl.pallas_call(
      _final_body,
      grid=(GRID_R,),
      in_specs=[
          pl.BlockSpec((RB, 16), lambda r: (r, 0)),
          pl.BlockSpec((2, 16), lambda r: (0, 0)),
          pl.BlockSpec((1, 2), lambda r: (0, 0)),
          pl.BlockSpec((1, 2), lambda r: (0, 0)),
      ],
      out_specs=pl.BlockSpec((RB, 2), lambda r: (r, 0)),
      out_shape=jax.ShapeDtypeStruct((N, 2), jnp.float32),
  )(s7, st7, g7, be7)


_CFG = [(1, 16), (1, 64), (1, 128), (2, 128), (1, 128), (1, 64), (1, 16),
        (1, 16)]


def kernel(x, edge_index,
           W0, b0, g0, be0, W1, b1, g1, be1, W2, b2, g2, be2,
           W3, b3, g3, be3, W4, b4, g4, be4, W5, b5, g5, be5,
           W6, b6, g6, be6, W7, b7, g7, be7):
  pad = EPAD - E
  pad_iota = jnp.arange(pad, dtype=jnp.int32)
  srcr = jnp.concatenate(
      [edge_index[0], pad_iota % N]).reshape(2, 16, NB, EB)
  dstr = jnp.concatenate(
      [edge_index[1], N + pad_iota % (NPAD - N)]).reshape(2, 16, NB, EB)

  W7p = jnp.pad(W7, ((0, 0), (0, 14)))
  b7p = jnp.pad(b7, (0, 14))

  degparts = _make_deg()(dstr)
  dinv = _dinv(degparts)

  Ws = [W0, W1, W2, W3, W4, W5, W6, W7p]
  bs = [b0, b1, b2, b3, b4, b5, b6, b7p]
  gs = [g0, g1, g2, g3, g4, g5, g6]
  bes = [be0, be1, be2, be3, be4, be5, be6]

  s_prev, st_prev = None, None
  xns = {}
  for i in range(8):
    nc, fc = _CFG[i]
    W = Ws[i]
    if i == 0:
      hp = _make_lin0(x.shape[1], W.shape[1], nc, fc)(x, W, dinv)
    elif i <= 4:
      want_xn = i in (1, 2, 3)
      k = W.shape[0]
      outs = _make_lin(k, W.shape[1], nc, fc, want_xn)(
          s_prev, st_prev, gs[i - 1].reshape(1, k), bes[i - 1].reshape(1, k),
          W, dinv)
      if want_xn:
        hp, xns[i - 1] = outs
      else:
        hp = outs[0]
    else:
      part = xns[7 - i]
      k1 = part.shape[1]
      wt, wb = W[:k1], W[k1:]
      hp = _make_lin_skip(k1, k1, W.shape[1], nc, fc)(
          s_prev, st_prev, gs[i - 1].reshape(1, k1),
          bes[i - 1].reshape(1, k1), part, wt, wb, dinv)
    parts = _make_spmm(nc)(hp, srcr, dstr)
    s_prev, st_prev = _make_combine(nc, fc)(
        parts, hp, dinv, bs[i].reshape(1, nc * fc))

  return _final(s_prev, st_prev, g7.reshape(1, 2), be7.reshape(1, 2))


# speedup vs baseline: 4.5369x; 1.0358x over previous
import functools

import jax
import jax.numpy as jnp
from jax import lax
from jax.experimental import pallas as pl
from jax.experimental.pallas import tpu as pltpu
from jax.experimental.pallas import tpu_sc as plsc

N = 10000
NPAD = 10240
E = 320000
NSUB = 32
EPAD = 327680
EB = 128
NB = 80
HB = NB // 2
STRIPE = NPAD // 16
RB = 2000
GRID_R = N // RB
EPS = 1e-5
SCW = 128


@functools.cache
def _make_spmm(nc):
  fc = SCW
  mesh = plsc.VectorSubcoreMesh(core_axis_name="c", subcore_axis_name="s")

  def body(h_hbm, srcr_hbm, dstr_hbm, out_hbm, acc, src_v, dst_v, rows0,
           rows1, sem0, sem1):
    c = lax.axis_index("c")
    s = lax.axis_index("s")

    def start_gather(ch, b, rows, sem):
      pltpu.make_async_copy(h_hbm.at[ch].at[src_v.at[b]], rows, sem).start()

    def finish_gather(ch, b, rows, sem):
      pltpu.make_async_copy(h_hbm.at[ch].at[src_v.at[b]], rows, sem).wait()
      pltpu.sync_copy(rows, acc.at[dst_v.at[b]], add=True)

    for ch in range(nc):
      def zrow(r, carry):
        for j in range(fc // 16):
          rows0[r, pl.ds(j * 16, 16)] = jnp.zeros((16,), jnp.float32)
        return carry
      lax.fori_loop(0, EB, zrow, 0)
      for t in range(STRIPE // EB):
        pltpu.sync_copy(rows0, acc.at[pl.ds(s * STRIPE + t * EB, EB)])
      plsc.subcore_barrier()

      for half in range(2):
        pltpu.sync_copy(srcr_hbm.at[c, s, pl.ds(half * HB, HB)], src_v)
        pltpu.sync_copy(dstr_hbm.at[c, s, pl.ds(half * HB, HB)], dst_v)

        start_gather(ch, 0, rows0, sem0)

        def edge_pair(i, carry):
          b = 2 * i
          start_gather(ch, b + 1, rows1, sem1)
          finish_gather(ch, b, rows0, sem0)
          start_gather(ch, b + 2, rows0, sem0)
          finish_gather(ch, b + 1, rows1, sem1)
          return carry

        lax.fori_loop(0, (HB - 2) // 2, edge_pair, 0)
        start_gather(ch, HB - 1, rows1, sem1)
        finish_gather(ch, HB - 2, rows0, sem0)
        finish_gather(ch, HB - 1, rows1, sem1)

      plsc.subcore_barrier()
      pltpu.sync_copy(acc.at[pl.ds(s * STRIPE, STRIPE)],
                      out_hbm.at[c, ch, pl.ds(s * STRIPE, STRIPE)])
      if ch + 1 < nc:
        plsc.subcore_barrier()

  return pl.kernel(
      body,
      out_type=jax.ShapeDtypeStruct((2, nc, NPAD, fc), jnp.float32),
      mesh=mesh,
      scratch_types=[
          pltpu.VMEM_SHARED((NPAD, fc), jnp.float32),
          pltpu.VMEM((HB, EB), jnp.int32),
          pltpu.VMEM((HB, EB), jnp.int32),
          pltpu.VMEM((EB, fc), jnp.float32),
          pltpu.VMEM((EB, fc), jnp.float32),
          pltpu.SemaphoreType.DMA,
          pltpu.SemaphoreType.DMA,
      ],
  )


@functools.cache
def _make_deg():
  mesh = plsc.VectorSubcoreMesh(core_axis_name="c", subcore_axis_name="s")

  def body(dstr_hbm, out_hbm, acc, dst_v, rows0):
    c = lax.axis_index("c")
    s = lax.axis_index("s")

    def fill(val):
      def frow(r, carry):
        for j in range(SCW // 16):
          rows0[r, pl.ds(j * 16, 16)] = jnp.full((16,), val, jnp.float32)
        return carry
      lax.fori_loop(0, EB, frow, 0)

    fill(0.0)
    for t in range(STRIPE // EB):
      pltpu.sync_copy(rows0, acc.at[pl.ds(s * STRIPE + t * EB, EB)])
    plsc.subcore_barrier()
    fill(1.0)

    for half in range(2):
      pltpu.sync_copy(dstr_hbm.at[c, s, pl.ds(half * HB, HB)], dst_v)

      def batch(b, carry):
        pltpu.sync_copy(rows0, acc.at[dst_v.at[b]], add=True)
        return carry
      lax.fori_loop(0, HB, batch, 0)

    plsc.subcore_barrier()
    pltpu.sync_copy(acc.at[pl.ds(s * STRIPE, STRIPE)],
                    out_hbm.at[c, pl.ds(s * STRIPE, STRIPE)])

  return pl.kernel(
      body,
      out_type=jax.ShapeDtypeStruct((2, NPAD, SCW), jnp.float32),
      mesh=mesh,
      scratch_types=[
          pltpu.VMEM_SHARED((NPAD, SCW), jnp.float32),
          pltpu.VMEM((HB, EB), jnp.int32),
          pltpu.VMEM((EB, SCW), jnp.float32),
      ],
  )


def _dinv_body(degp_ref, dinv_ref):
  deg = degp_ref[0, :, 0:1] + degp_ref[1, :, 0:1] + 1.0
  dinv_ref[...] = lax.rsqrt(deg)


def _dinv(degparts):
  return pl.pallas_call(
      _dinv_body,
      grid=(GRID_R,),
      in_specs=[pl.BlockSpec((2, RB, SCW), lambda r: (0, r, 0))],
      out_specs=pl.BlockSpec((RB, 1), lambda r: (r, 0)),
      out_shape=jax.ShapeDtypeStruct((N, 1), jnp.float32),
  )(degparts)


def _bn_from_stats(s, stats, g, be):
  inv_n = 1.0 / N
  m = stats[0:1, :] * inv_n
  v = stats[1:2, :] * inv_n - m * m
  return g * (s - m) * lax.rsqrt(v + EPS) + be


def _store_chunks(h_ref, hp, nc, fc):
  for c in range(nc):
    chunk = hp[:, c * fc:(c + 1) * fc]
    if fc < SCW:
      chunk = jnp.pad(chunk, ((0, 0), (0, SCW - fc)))
    h_ref[c] = chunk


@functools.cache
def _make_lin0(k, f, nc, fc):
  def body(x_ref, w_ref, dinv_ref, h_ref):
    hp = dinv_ref[...] * jnp.dot(x_ref[...], w_ref[...],
                                 preferred_element_type=jnp.float32)
    _store_chunks(h_ref, hp, nc, fc)

  return pl.pallas_call(
      body,
      grid=(GRID_R,),
      in_specs=[
          pl.BlockSpec((RB, k), lambda r: (r, 0)),
          pl.BlockSpec((k, f), lambda r: (0, 0)),
          pl.BlockSpec((RB, 1), lambda r: (r, 0)),
      ],
      out_specs=pl.BlockSpec((nc, RB, SCW), lambda r: (0, r, 0)),
      out_shape=jax.ShapeDtypeStruct((nc, N, SCW), jnp.float32),
  )


@functools.cache
def _make_lin(k, f, nc, fc, want_xn):
  def body(s_ref, st_ref, g_ref, be_ref, w_ref, dinv_ref, h_ref, *xn_ref):
    xn = _bn_from_stats(s_ref[...], st_ref[...], g_ref[...], be_ref[...])
    hp = dinv_ref[...] * jnp.dot(xn, w_ref[...],
                                 preferred_element_type=jnp.float32)
    _store_chunks(h_ref, hp, nc, fc)
    if want_xn:
      xn_ref[0][...] = xn

  out_specs = [pl.BlockSpec((nc, RB, SCW), lambda r: (0, r, 0))]
  out_shape = [jax.ShapeDtypeStruct((nc, N, SCW), jnp.float32)]
  if want_xn:
    out_specs.append(pl.BlockSpec((RB, k), lambda r: (r, 0)))
    out_shape.append(jax.ShapeDtypeStruct((N, k), jnp.float32))
  return pl.pallas_call(
      body,
      grid=(GRID_R,),
      in_specs=[
          pl.BlockSpec((RB, k), lambda r: (r, 0)),
          pl.BlockSpec((2, k), lambda r: (0, 0)),
          pl.BlockSpec((1, k), lambda r: (0, 0)),
          pl.BlockSpec((1, k), lambda r: (0, 0)),
          pl.BlockSpec((k, f), lambda r: (0, 0)),
          pl.BlockSpec((RB, 1), lambda r: (r, 0)),
      ],
      out_specs=out_specs,
      out_shape=out_shape,
  )


@functools.cache
def _make_lin_skip(k1, k2, f, nc, fc):
  def body(s_ref, st_ref, g_ref, be_ref, part_ref, wt_ref, wb_ref, dinv_ref,
           h_ref):
    xn = _bn_from_stats(s_ref[...], st_ref[...], g_ref[...], be_ref[...])
    hp = jnp.dot(xn, wt_ref[...], preferred_element_type=jnp.float32)
    hp = hp + jnp.dot(part_ref[...], wb_ref[...],
                      preferred_element_type=jnp.float32)
    hp = dinv_ref[...] * hp
    _store_chunks(h_ref, hp, nc, fc)

  return pl.pallas_call(
      body,
      grid=(GRID_R,),
      in_specs=[
          pl.BlockSpec((RB, k1), lambda r: (r, 0)),
          pl.BlockSpec((2, k1), lambda r: (0, 0)),
          pl.BlockSpec((1, k1), lambda r: (0, 0)),
          pl.BlockSpec((1, k1), lambda r: (0, 0)),
          pl.BlockSpec((RB, k2), lambda r: (r, 0)),
          pl.BlockSpec((k1, f), lambda r: (0, 0)),
          pl.BlockSpec((k2, f), lambda r: (0, 0)),
          pl.BlockSpec((RB, 1), lambda r: (r, 0)),
      ],
      out_specs=pl.BlockSpec((nc, RB, SCW), lambda r: (0, r, 0)),
      out_shape=jax.ShapeDtypeStruct((nc, N, SCW), jnp.float32),
  )


@functools.cache
def _make_combine(nc, fc):
  def body(p_ref, h_ref, dinv_ref, b_ref, s_ref, st_ref):
    r = pl.program_id(1)
    z = (dinv_ref[...] *
         (p_ref[0, 0, :, :fc] + p_ref[1, 0, :, :fc] + h_ref[0, :, :fc])
         + b_ref[...])
    sv = jax.nn.sigmoid(z)
    s_ref[...] = sv

    @pl.when(r == 0)
    def _():
      st_ref[...] = jnp.zeros_like(st_ref)

    st_ref[0:1, :] += jnp.sum(sv, axis=0, keepdims=True)
    st_ref[1:2, :] += jnp.sum(sv * sv, axis=0, keepdims=True)

  f = nc * fc
  return pl.pallas_call(
      body,
      grid=(nc, GRID_R),
      in_specs=[
          pl.BlockSpec((2, 1, RB, SCW), lambda c, r: (0, c, r, 0)),
          pl.BlockSpec((1, RB, SCW), lambda c, r: (c, r, 0)),
          pl.BlockSpec((RB, 1), lambda c, r: (r, 0)),
          pl.BlockSpec((1, fc), lambda c, r: (0, c)),
      ],
      out_specs=[
          pl.BlockSpec((RB, fc), lambda c, r: (r, c)),
          pl.BlockSpec((2, fc), lambda c, r: (0, c)),
      ],
      out_shape=[
          jax.ShapeDtypeStruct((N, f), jnp.float32),
          jax.ShapeDtypeStruct((2, f), jnp.float32),
      ],
  )


def _final_body(s_ref, st_ref, g_ref, be_ref, out_ref):
  xn = _bn_from_stats(s_ref[:, 0:2], st_ref[:, 0:2], g_ref[...], be_ref[...])
  mx = jnp.max(xn, axis=1, keepdims=True)
  e = jnp.exp(xn - mx)
  out_ref[...] = e / jnp.sum(e, axis=1, keepdims=True)


def _final(s7, st7, g7, be7):
  return pl.pallas_call(
      _final_body,
      grid=(GRID_R,),
      in_specs=[
          pl.BlockSpec((RB, 16), lambda r: (r, 0)),
          pl.BlockSpec((2, 16), lambda r: (0, 0)),
          pl.BlockSpec((1, 2), lambda r: (0, 0)),
          pl.BlockSpec((1, 2), lambda r: (0, 0)),
      ],
      out_specs=pl.BlockSpec((RB, 2), lambda r: (r, 0)),
      out_shape=jax.ShapeDtypeStruct((N, 2), jnp.float32),
  )(s7, st7, g7, be7)


_CFG = [(1, 16), (1, 64), (1, 128), (2, 128), (1, 128), (1, 64), (1, 16),
        (1, 16)]


def kernel(x, edge_index,
           W0, b0, g0, be0, W1, b1, g1, be1, W2, b2, g2, be2,
           W3, b3, g3, be3, W4, b4, g4, be4, W5, b5, g5, be5,
           W6, b6, g6, be6, W7, b7, g7, be7):
  pad = EPAD - E
  pad_iota = jnp.arange(pad, dtype=jnp.int32)
  srcr = jnp.concatenate(
      [edge_index[0], pad_iota % N]).reshape(2, 16, NB, EB)
  dstr = jnp.concatenate(
      [edge_index[1], N + pad_iota % (NPAD - N)]).reshape(2, 16, NB, EB)

  W7p = jnp.pad(W7, ((0, 0), (0, 14)))
  b7p = jnp.pad(b7, (0, 14))

  degparts = _make_deg()(dstr)
  dinv = _dinv(degparts)

  Ws = [W0, W1, W2, W3, W4, W5, W6, W7p]
  bs = [b0, b1, b2, b3, b4, b5, b6, b7p]
  gs = [g0, g1, g2, g3, g4, g5, g6]
  bes = [be0, be1, be2, be3, be4, be5, be6]

  s_prev, st_prev = None, None
  xns = {}
  for i in range(8):
    nc, fc = _CFG[i]
    W = Ws[i]
    if i == 0:
      hp = _make_lin0(x.shape[1], W.shape[1], nc, fc)(x, W, dinv)
    elif i <= 4:
      want_xn = i in (1, 2, 3)
      k = W.shape[0]
      outs = _make_lin(k, W.shape[1], nc, fc, want_xn)(
          s_prev, st_prev, gs[i - 1].reshape(1, k), bes[i - 1].reshape(1, k),
          W, dinv)
      if want_xn:
        hp, xns[i - 1] = outs
      else:
        hp = outs[0]
    else:
      part = xns[7 - i]
      k1 = part.shape[1]
      wt, wb = W[:k1], W[k1:]
      hp = _make_lin_skip(k1, k1, W.shape[1], nc, fc)(
          s_prev, st_prev, gs[i - 1].reshape(1, k1),
          bes[i - 1].reshape(1, k1), part, wt, wb, dinv)
    parts = _make_spmm(nc)(hp, srcr, dstr)
    s_prev, st_prev = _make_combine(nc, fc)(
        parts, hp, dinv, bs[i].reshape(1, nc * fc))

  return _final(s_prev, st_prev, g7.reshape(1, 2), be7.reshape(1, 2))
